# Initial kernel scaffold; baseline (speedup 1.0000x reference)
import jax
import jax.numpy as jnp
from jax.experimental import pallas as pl


def kernel(x_G, x_R, edge_index_GR, edge_index_RR, Wk, bk, Wq, bq, Wv, bv, Ws, bs, Wg, bg, W_las):
    raise NotImplementedError("write your pallas kernel here")


# speedup vs baseline: 1.7617x; 1.7617x over previous
import functools

import jax
import jax.numpy as jnp
from jax import lax
from jax.experimental import pallas as pl
from jax.experimental.pallas import tpu as pltpu
from jax.experimental.pallas import tpu_sc as plsc

NR = 10000
GE = 128
RE = 256
H = 128
E = 160000
NSUB = 16
NCORE = 2
RPT = 624

_MESH = plsc.VectorSubcoreMesh(core_axis_name="c", subcore_axis_name="s")

CD = 80
CE = 80


def _fill_rows(ref, nrows, width, value):
    vec = jnp.full((16,), value, jnp.float32)

    def body(i, carry):
        for f in range(width // 16):
            ref[i, pl.ds(f * 16, 16)] = vec
        return carry

    lax.fori_loop(0, nrows, body, 0)


def _zero_acc(zbuf, acc, s, zrows):
    base = s * RPT
    nfull, rem = RPT // zrows, RPT % zrows
    for r in range(nfull):
        pltpu.sync_copy(zbuf, acc.at[pl.ds(base + r * zrows, zrows)])
    if rem:
        pltpu.sync_copy(zbuf.at[pl.ds(0, rem)],
                        acc.at[pl.ds(base + nfull * zrows, rem)])

    @pl.when(s == 0)
    def _():
        pltpu.sync_copy(zbuf.at[pl.ds(0, 16)], acc.at[pl.ds(NR - 16, 16)])


def _drain_acc(acc, out_hbm, s, c):
    base = s * RPT
    pltpu.sync_copy(acc.at[pl.ds(base, RPT)],
                    out_hbm.at[pl.ds(c * NR + base, RPT)])

    @pl.when(s == 0)
    def _():
        pltpu.sync_copy(acc.at[pl.ds(NR - 16, 16)],
                        out_hbm.at[pl.ds(c * NR + NR - 16, 16)])


@functools.partial(
    pl.kernel,
    out_type=jax.ShapeDtypeStruct((NCORE * NR, H), jnp.float32),
    mesh=_MESH,
    scratch_types=[
        pltpu.VMEM_SHARED((NR, H), jnp.float32),
        pltpu.VMEM((CD,), jnp.int32),
        pltpu.VMEM((CD, H), jnp.float32),
        pltpu.VMEM((CD, H), jnp.float32),
    ],
)
def _sc_degree(dst_hbm, out_hbm, deg_sh, idx_v, ones_v, zbuf):
    c = lax.axis_index("c")
    s = lax.axis_index("s")
    _fill_rows(zbuf, CD, H, 0.0)
    _fill_rows(ones_v, CD, H, 1.0)
    _zero_acc(zbuf, deg_sh, s, CD)
    plsc.subcore_barrier()
    ept = E // NSUB
    base = s * ept

    def chunk(k, carry):
        pltpu.sync_copy(dst_hbm.at[pl.ds(base + k * CD, CD)], idx_v)
        pltpu.sync_copy(ones_v, deg_sh.at[idx_v], add=True)
        return carry

    lax.fori_loop(0, ept // CD, chunk, 0)
    plsc.subcore_barrier()
    _drain_acc(deg_sh, out_hbm, s, c)


@functools.partial(
    pl.kernel,
    out_type=jax.ShapeDtypeStruct((NCORE * NR, H), jnp.float32),
    mesh=_MESH,
    scratch_types=[
        pltpu.VMEM_SHARED((NR, H), jnp.float32),
        pltpu.VMEM((CE,), jnp.int32),
        pltpu.VMEM((CE,), jnp.int32),
        pltpu.VMEM((CE,), jnp.int32),
        pltpu.VMEM((CE, H), jnp.float32),
        pltpu.VMEM((CE, 2 * H), jnp.float32),
        pltpu.VMEM((CE, H), jnp.float32),
        pltpu.SemaphoreType.DMA,
        pltpu.SemaphoreType.DMA,
    ],
)
def _sc_resgated(k_hbm, qv_hbm, dst_hbm, src_hbm, out_hbm,
                 acc, dstv, dstv_adj, srcv_adj, krows, qvrows, msg,
                 sem1, sem2):
    c = lax.axis_index("c")
    s = lax.axis_index("s")
    _fill_rows(msg, CE, H, 0.0)
    _zero_acc(msg, acc, s, CE)
    plsc.subcore_barrier()
    ept = E // NSUB
    base = s * ept
    coff = c * NR

    def chunk(k, carry):
        off = base + k * CE
        pltpu.sync_copy(dst_hbm.at[pl.ds(off, CE)], dstv)
        pltpu.sync_copy(src_hbm.at[pl.ds(off, CE)], srcv_adj)
        for j in range(CE // 16):
            sl = pl.ds(j * 16, 16)
            dstv_adj[sl] = dstv[sl] + coff
            srcv_adj[sl] = srcv_adj[sl] + coff
        cp1 = pltpu.async_copy(k_hbm.at[dstv_adj], krows, sem1)
        cp2 = pltpu.async_copy(qv_hbm.at[srcv_adj], qvrows, sem2)
        cp1.wait()
        cp2.wait()

        def edge(i, icarry):
            for f in range(H // 16):
                sl = pl.ds(f * 16, 16)
                kv = krows[i, sl]
                qv = qvrows[i, sl]
                vv = qvrows[i, pl.ds(H + f * 16, 16)]
                eta = 1.0 / (1.0 + jnp.exp(-(kv + qv)))
                msg[i, sl] = eta * vv
            return icarry

        lax.fori_loop(0, CE, edge, 0)
        pltpu.sync_copy(msg, acc.at[dstv], add=True)
        return carry

    lax.fori_loop(0, ept // CE, chunk, 0)
    plsc.subcore_barrier()
    _drain_acc(acc, out_hbm, s, c)


@functools.partial(
    pl.kernel,
    out_type=jax.ShapeDtypeStruct((NCORE * NR, H), jnp.float32),
    mesh=_MESH,
    scratch_types=[
        pltpu.VMEM_SHARED((NR, H), jnp.float32),
        pltpu.VMEM((CE,), jnp.int32),
        pltpu.VMEM((CE,), jnp.int32),
        pltpu.VMEM((CE, H), jnp.float32),
        pltpu.SemaphoreType.DMA,
    ],
)
def _sc_gcn(xw_hbm, dst_hbm, src_hbm, out_hbm, acc, dstv, srcv_adj, rows, sem1):
    c = lax.axis_index("c")
    s = lax.axis_index("s")
    _fill_rows(rows, CE, H, 0.0)
    _zero_acc(rows, acc, s, CE)
    plsc.subcore_barrier()
    ept = E // NSUB
    base = s * ept
    coff = c * NR

    def chunk(k, carry):
        off = base + k * CE
        pltpu.sync_copy(dst_hbm.at[pl.ds(off, CE)], dstv)
        pltpu.sync_copy(src_hbm.at[pl.ds(off, CE)], srcv_adj)
        for j in range(CE // 16):
            sl = pl.ds(j * 16, 16)
            srcv_adj[sl] = srcv_adj[sl] + coff
        pltpu.async_copy(xw_hbm.at[srcv_adj], rows, sem1).wait()
        pltpu.sync_copy(rows, acc.at[dstv], add=True)
        return carry

    lax.fori_loop(0, ept // CE, chunk, 0)
    plsc.subcore_barrier()
    _drain_acc(acc, out_hbm, s, c)


NB = 10
BN = NR // NB


def _dinv_block(deg_ref):
    deg = 1.0 + (1.0 / (2 * H)) * jnp.sum(jnp.sum(deg_ref[...], axis=0), axis=1)
    return lax.rsqrt(deg)[:, None]


def _tc_pre(xg, xr, deg16, Wk, bk, Wq, bq, Wv, bv, Ws, bs, Wg):
    def body(xg_ref, xr_ref, deg_ref, wk_ref, bk_ref, wq_ref, bq_ref,
             wv_ref, bv_ref, ws_ref, bs_ref, wg_ref,
             k_ref, qv0_ref, qv1_ref, xwp_ref, s_ref):
        xgb = xg_ref[...]
        xrb = xr_ref[...]
        dinv = _dinv_block(deg_ref)
        q0 = xgb @ wq_ref[0] + bq_ref[0, 0][None, :]
        v0 = xgb @ wv_ref[0] + bv_ref[0, 0][None, :]
        q1 = xgb @ wq_ref[1] + bq_ref[1, 0][None, :]
        v1 = xgb @ wv_ref[1] + bv_ref[1, 0][None, :]
        qv0_ref[0] = jnp.concatenate([q0, v0], axis=1)
        qv1_ref[0] = jnp.concatenate([q1, v1], axis=1)
        k_ref[0] = xrb @ wk_ref[0] + bk_ref[0, 0][None, :]
        xwp_ref[0] = (xrb @ wg_ref[0]) * dinv
        s_ref[...] = xrb @ ws_ref[0] + bs_ref[0, 0][None, :]

    return pl.pallas_call(
        body,
        grid=(NB, NCORE),
        in_specs=[
            pl.BlockSpec((BN, GE), lambda i,ic: (i, 0)),
            pl.BlockSpec((BN, RE), lambda i, ic: (i, 0)),
            pl.BlockSpec((NCORE, BN, H), lambda i, ic: (0, i, 0)),
            pl.BlockSpec((1, RE, H), lambda i, ic: (0, 0, ic)),
            pl.BlockSpec((1, 8, H), lambda i, ic: (0, 0, ic)),
            pl.BlockSpec((2, GE, H), lambda i, ic: (0, 0, ic)),
            pl.BlockSpec((2, 8, H), lambda i, ic: (0, 0, ic)),
            pl.BlockSpec((2, GE, H), lambda i, ic: (0, 0, ic)),
            pl.BlockSpec((2, 8, H), lambda i, ic: (0, 0, ic)),
            pl.BlockSpec((1, RE, H), lambda i, ic: (0, 0, ic)),
            pl.BlockSpec((1, 8, H), lambda i, ic: (0, 0, ic)),
            pl.BlockSpec((1, RE, H), lambda i, ic: (0, 0, ic)),
        ],
        out_specs=[
            pl.BlockSpec((1, BN, H), lambda i, ic: (ic, i, 0)),
            pl.BlockSpec((1, BN, 2 * H), lambda i, ic: (ic, i, 0)),
            pl.BlockSpec((1, BN, 2 * H), lambda i, ic: (ic, i, 0)),
            pl.BlockSpec((1, BN, H), lambda i, ic: (ic, i, 0)),
            pl.BlockSpec((BN, H), lambda i, ic: (i, ic)),
        ],
        out_shape=[
            jax.ShapeDtypeStruct((NCORE, NR, H), jnp.float32),
            jax.ShapeDtypeStruct((NCORE, NR, 2 * H), jnp.float32),
            jax.ShapeDtypeStruct((NCORE, NR, 2 * H), jnp.float32),
            jax.ShapeDtypeStruct((NCORE, NR, H), jnp.float32),
            jax.ShapeDtypeStruct((NR, RE), jnp.float32),
        ],
    )(xg, xr, deg16, Wk, bk, Wq, bq, Wv, bv, Ws, bs, Wg)


def _combine(g_ref, r_ref, xwp_ref, s_ref, deg_ref, bg_ref):
    dinv = _dinv_block(deg_ref)
    gfull = jnp.concatenate([g_ref[0], g_ref[1]], axis=1)
    rfull = jnp.concatenate([r_ref[0], r_ref[1]], axis=1)
    xwpfull = jnp.concatenate([xwp_ref[0], xwp_ref[1]], axis=1)
    pre = gfull + s_ref[...] + dinv * (rfull + xwpfull) + bg_ref[0, 0][None, :]
    gelu = 0.5 * pre * (1.0 + lax.erf(pre * (1.0 / jnp.sqrt(2.0).astype(jnp.float32))))
    return gelu, dinv


def _tc_mid(G0, R0, XWP0, S0, deg16, Wk, bk, Ws, bs, Wg, bg):
    def body(g_ref, r_ref, xwp_ref, s_ref, deg_ref, bg_ref,
             wk_ref, bk_ref, ws_ref, bs_ref, wg_ref,
             k1_ref, xwp1_ref, s1_ref):
        xr1, dinv = _combine(g_ref, r_ref, xwp_ref, s_ref, deg_ref, bg_ref)
        k1 = xr1 @ wk_ref[0] + bk_ref[0, 0][None, :]
        k1_ref[0] = k1[:, :H]
        k1_ref[1] = k1[:, H:]
        xwp1 = (xr1 @ wg_ref[0]) * dinv
        xwp1_ref[0] = xwp1[:, :H]
        xwp1_ref[1] = xwp1[:, H:]
        s1_ref[...] = xr1 @ ws_ref[0] + bs_ref[0, 0][None, :]

    return pl.pallas_call(
        body,
        grid=(NB,),
        in_specs=[
            pl.BlockSpec((NCORE, BN, H), lambda i: (0, i, 0)),
            pl.BlockSpec((NCORE, BN, H), lambda i: (0, i, 0)),
            pl.BlockSpec((NCORE, BN, H), lambda i: (0, i, 0)),
            pl.BlockSpec((BN, RE), lambda i: (i, 0)),
            pl.BlockSpec((NCORE, BN, H), lambda i: (0, i, 0)),
            pl.BlockSpec((1, 8, RE), lambda i: (0, 0, 0)),
            pl.BlockSpec((1, RE, RE), lambda i: (1, 0, 0)),
            pl.BlockSpec((1, 8, RE), lambda i: (1, 0, 0)),
            pl.BlockSpec((1, RE, RE), lambda i: (1, 0, 0)),
            pl.BlockSpec((1, 8, RE), lambda i: (1, 0, 0)),
            pl.BlockSpec((1, RE, RE), lambda i: (1, 0, 0)),
        ],
        out_specs=[
            pl.BlockSpec((NCORE, BN, H), lambda i: (0, i, 0)),
            pl.BlockSpec((NCORE, BN, H), lambda i: (0, i, 0)),
            pl.BlockSpec((BN, RE), lambda i: (i, 0)),
        ],
        out_shape=[
            jax.ShapeDtypeStruct((NCORE, NR, H), jnp.float32),
            jax.ShapeDtypeStruct((NCORE, NR, H), jnp.float32),
            jax.ShapeDtypeStruct((NR, RE), jnp.float32),
        ],
    )(G0, R0, XWP0, S0, deg16, bg, Wk, bk, Ws, bs, Wg)


def _tc_post(G1, R1, XWP1, S1, deg16, bg, W_las):
    def body(g_ref, r_ref, xwp_ref, s_ref, deg_ref, bg_ref, wl_ref, out_ref):
        xr2, _ = _combine(g_ref, r_ref, xwp_ref, s_ref, deg_ref, bg_ref)
        t1 = xr2[:, :H]
        t2 = xr2[:, H:]
        wl = wl_ref[...]
        z = (lax.dot_general(t1, wl, (((1,), (0,)), ((), ())))
             - lax.dot_general(t1, wl, (((1,), (1,)), ((), ()))))
        out_ref[...] = jnp.sum(z * t2, axis=1, keepdims=True)

    return pl.pallas_call(
        body,
        grid=(NB,),
        in_specs=[
            pl.BlockSpec((NCORE, BN, H), lambda i: (0, i, 0)),
            pl.BlockSpec((NCORE, BN, H), lambda i: (0, i, 0)),
            pl.BlockSpec((NCORE, BN, H), lambda i: (0, i, 0)),
            pl.BlockSpec((BN, RE), lambda i: (i, 0)),
            pl.BlockSpec((NCORE, BN, H), lambda i: (0, i, 0)),
            pl.BlockSpec((1, 8, RE), lambda i: (1, 0, 0)),
            pl.BlockSpec((H, H), lambda i: (0, 0)),
        ],
        out_specs=pl.BlockSpec((BN, 1), lambda i: (i, 0)),
        out_shape=jax.ShapeDtypeStruct((NR, 1), jnp.float32),
    )(G1, R1, XWP1, S1, deg16, bg, W_las)


def kernel(x_G, x_R, edge_index_GR, edge_index_RR,
           Wk, bk, Wq, bq, Wv, bv, Ws, bs, Wg, bg, W_las):
    xg = x_G[0]
    xr = x_R[0]
    src_gr = edge_index_GR[0]
    dst_gr = edge_index_GR[1]
    src_rr = edge_index_RR[0]
    dst_rr = edge_index_RR[1]
    bk8 = jnp.broadcast_to(bk[:, None, :], (2, 8, RE))
    bq8 = jnp.broadcast_to(bq[:, None, :], (2, 8, RE))
    bv8 = jnp.broadcast_to(bv[:, None, :], (2, 8, RE))
    bs8 = jnp.broadcast_to(bs[:, None, :], (2, 8, RE))
    bg8 = jnp.broadcast_to(bg[:, None, :], (2, 8, RE))

    deg16 = _sc_degree(dst_rr).reshape(NCORE, NR, H)
    K0, QV0, QV1, XWP0, S0 = _tc_pre(xg, xr, deg16,
                                     Wk, bk8, Wq, bq8, Wv, bv8, Ws, bs8, Wg)
    G0 = _sc_resgated(K0.reshape(NCORE * NR, H),
                      QV0.reshape(NCORE * NR, 2 * H),
                      dst_gr, src_gr).reshape(NCORE, NR, H)
    R0 = _sc_gcn(XWP0.reshape(NCORE * NR, H),
                 dst_rr, src_rr).reshape(NCORE, NR, H)
    K1, XWP1, S1 = _tc_mid(G0, R0, XWP0, S0, deg16, Wk, bk8, Ws, bs8, Wg, bg8)
    G1 = _sc_resgated(K1.reshape(NCORE * NR, H),
                      QV1.reshape(NCORE * NR, 2 * H),
                      dst_gr, src_gr).reshape(NCORE, NR, H)
    R1 = _sc_gcn(XWP1.reshape(NCORE * NR, H),
                 dst_rr, src_rr).reshape(NCORE, NR, H)
    out = _tc_post(G1, R1, XWP1, S1, deg16, bg8, W_las)
    return out.reshape(NR)
